# Initial kernel scaffold; baseline (speedup 1.0000x reference)
import jax
import jax.numpy as jnp
from jax.experimental import pallas as pl


def kernel(x, adj, Wl, bl, Wr, br, att, gat_bias, W1, b1, W2, b2, g1, be1, g2, be2):
    raise NotImplementedError("write your pallas kernel here")


# speedup vs baseline: 614.8311x; 614.8311x over previous
import functools

import jax
import jax.numpy as jnp
from jax.experimental import pallas as pl
from jax.experimental.pallas import tpu as pltpu

_NEG = -1e30


def _layer_norm(v, g, b, eps=1e-5):
    mu = jnp.mean(v, axis=-1, keepdims=True)
    d = v - mu
    var = jnp.mean(d * d, axis=-1, keepdims=True)
    return d * jax.lax.rsqrt(var + eps) * g + b


def _encoder_kernel(H, x_ref, adj_ref, Wl_ref, bl_ref, Wr_ref, br_ref, att_ref,
                    gb_ref, W1_ref, b1_ref, W2_ref, b2_ref, g1_ref, be1_ref,
                    g2_ref, be2_ref, out_ref):
    L, _, _ = Wl_ref.shape
    N = x_ref.shape[1]
    D = x_ref.shape[2]
    C = D // H

    mask = adj_ref[0] != 0
    h = x_ref[0]

    for l in range(L):
        xl = jnp.dot(h, Wl_ref[l], preferred_element_type=jnp.float32) + bl_ref[l]
        xr = jnp.dot(h, Wr_ref[l], preferred_element_type=jnp.float32) + br_ref[l]
        xr_t = xr.T
        head_outs = []
        for hh in range(H):
            acc = jnp.zeros((N, N), jnp.float32)
            for c in range(C):
                k = hh * C + c
                t = xl[:, k:k + 1] + xr_t[k:k + 1, :]
                acc = acc + att_ref[l, 0, k] * jnp.maximum(t, 0.2 * t)
            emax = jnp.max(jnp.where(mask, acc, _NEG), axis=0, keepdims=True)
            p = jnp.where(mask, jnp.exp(acc - emax), 0.0)
            denom = jnp.sum(p, axis=0, keepdims=True)
            scale = jnp.where(denom > 0, 1.0 / denom, 1.0).reshape(N, 1)
            num = jax.lax.dot_general(p, xl[:, hh * C:(hh + 1) * C],
                                      (((0,), (0,)), ((), ())),
                                      preferred_element_type=jnp.float32)
            head_outs.append(num * scale)
        gat = jnp.concatenate(head_outs, axis=1) + gb_ref[l]

        y = _layer_norm(gat, g1_ref[l], be1_ref[l])
        y = jnp.dot(y, W1_ref[l], preferred_element_type=jnp.float32) + b1_ref[l]
        y = y * jax.nn.sigmoid(y)
        y = jnp.dot(y, W2_ref[l], preferred_element_type=jnp.float32) + b2_ref[l]
        h = _layer_norm(gat + y, g2_ref[l], be2_ref[l])

    out_ref[0] = h


def kernel(x, adj, Wl, bl, Wr, br, att, gat_bias, W1, b1, W2, b2, g1, be1, g2, be2):
    B, N, D = x.shape
    L = Wl.shape[0]
    H = att.shape[1]
    FF = W1.shape[2]

    bl2 = bl.reshape(L, 1, D)
    br2 = br.reshape(L, 1, D)
    att2 = att.reshape(L, 1, D)
    gb2 = gat_bias.reshape(L, 1, D)
    b12 = b1.reshape(L, 1, FF)
    b22 = b2.reshape(L, 1, D)
    g12 = g1.reshape(L, 1, D)
    be12 = be1.reshape(L, 1, D)
    g22 = g2.reshape(L, 1, D)
    be22 = be2.reshape(L, 1, D)

    full = lambda shape: pl.BlockSpec(shape, lambda b: (0,) * len(shape))
    out = pl.pallas_call(
        functools.partial(_encoder_kernel, H),
        grid=(B,),
        in_specs=[
            pl.BlockSpec((1, N, D), lambda b: (b, 0, 0)),
            pl.BlockSpec((1, N, N), lambda b: (b, 0, 0)),
            full((L, D, D)), full((L, 1, D)),
            full((L, D, D)), full((L, 1, D)),
            full((L, 1, D)), full((L, 1, D)),
            full((L, D, FF)), full((L, 1, FF)),
            full((L, FF, D)), full((L, 1, D)),
            full((L, 1, D)), full((L, 1, D)),
            full((L, 1, D)), full((L, 1, D)),
        ],
        out_specs=pl.BlockSpec((1, N, D), lambda b: (b, 0, 0)),
        out_shape=jax.ShapeDtypeStruct((B, N, D), jnp.float32),
        compiler_params=pltpu.CompilerParams(
            dimension_semantics=("parallel",)),
    )(x, adj, Wl, bl2, Wr, br2, att2, gb2, W1, b12, W2, b22,
      g12, be12, g22, be22)
    return out
